# baseline (speedup 1.0000x reference)
import functools

import jax
import jax.numpy as jnp
from jax import lax
from jax.experimental import pallas as pl
from jax.experimental.pallas import tpu as pltpu
from jax.experimental.pallas import tpu_sc as plsc

_FEAT = 64
_BATCH = 16384
_CLASSES = 100000
_NC, _NS, _L = 2, 16, 16
_NW = _NC * _NS
_RPW = _FEAT // _NW
_NQ = 4
_QB = _BATCH // _NQ
_UNROLL = 16


@functools.partial(
    pl.kernel,
    out_type=jax.ShapeDtypeStruct((_NW, _L), jnp.float32),
    mesh=plsc.VectorSubcoreMesh(core_axis_name="c", subcore_axis_name="s"),
    compiler_params=pltpu.CompilerParams(needs_layout_passes=False),
    scratch_types=[
        pltpu.VMEM((_CLASSES,), jnp.float32),
        pltpu.VMEM((_BATCH,), jnp.int32),
        pltpu.VMEM((_QB,), jnp.float32),
        pltpu.VMEM((_QB,), jnp.float32),
        pltpu.VMEM((_L,), jnp.float32),
        pltpu.SemaphoreType.DMA,
        pltpu.SemaphoreType.DMA,
        pltpu.SemaphoreType.DMA,
        pltpu.SemaphoreType.DMA,
    ],
)
def _center_partials(feat_hbm, lab_hbm, cent_hbm, out_hbm,
                     crow, labv, fq0, fq1, acc_v,
                     csem, fsem0, fsem1, lsem):
    wid = lax.axis_index("s") * _NC + lax.axis_index("c")
    j0 = wid * _RPW
    fqs = (fq0, fq1)
    fsems = (fsem0, fsem1)

    def issue_feat(r, q):
        return pltpu.async_copy(feat_hbm.at[j0 + r, pl.ds(q * _QB, _QB)],
                                fqs[q % 2], fsems[q % 2])

    lab_h = pltpu.async_copy(lab_hbm, labv, lsem)
    crow_h = pltpu.async_copy(cent_hbm.at[j0], crow, csem)
    pending = {(0, 0): issue_feat(0, 0), (0, 1): issue_feat(0, 1)}
    lab_h.wait()

    accs = tuple(jnp.zeros((_L,), jnp.float32) for _ in range(_UNROLL))
    for r in range(_RPW):
        crow_h.wait()
        for q in range(_NQ):
            fq = fqs[q % 2]
            pending.pop((r, q)).wait()

            def step(g, a, q=q, fq=fq):
                out = []
                for u in range(_UNROLL):
                    o = g * _L * _UNROLL + u * _L
                    idx = labv[pl.ds(q * _QB + o, _L)]
                    f = fq[pl.ds(o, _L)]
                    c = plsc.load_gather(crow, [idx])
                    d = f - c
                    out.append(a[u] + d * d)
                return tuple(out)

            accs = lax.fori_loop(0, _QB // (_L * _UNROLL), step, accs)

            if q + 2 < _NQ:
                pending[(r, q + 2)] = issue_feat(r, q + 2)
            elif r + 1 < _RPW:
                pending[(r + 1, q - 2)] = issue_feat(r + 1, q - 2)

        if r + 1 < _RPW:
            crow_h = pltpu.async_copy(cent_hbm.at[j0 + r + 1], crow, csem)

    acc_v[...] = functools.reduce(lambda x, y: x + y, accs)
    pltpu.sync_copy(acc_v, out_hbm.at[wid])


def kernel(features, labels, centers):
    lab = labels.astype(jnp.int32)
    partials = _center_partials(features.T, lab, centers.T)
    return jnp.sum(partials) / features.shape[0]


# speedup vs baseline: 1.0318x; 1.0318x over previous
import functools

import jax
import jax.numpy as jnp
from jax import lax
from jax.experimental import pallas as pl
from jax.experimental.pallas import tpu as pltpu
from jax.experimental.pallas import tpu_sc as plsc

_FEAT = 64
_BATCH = 16384
_CLASSES = 100000
_NC, _NS, _L = 2, 16, 16
_NW = _NC * _NS
_RPW = _FEAT // _NW
_NQ = 4
_QB = _BATCH // _NQ
_UNROLL = 8


@functools.partial(
    pl.kernel,
    out_type=jax.ShapeDtypeStruct((_NW, _L), jnp.float32),
    mesh=plsc.VectorSubcoreMesh(core_axis_name="c", subcore_axis_name="s"),
    compiler_params=pltpu.CompilerParams(needs_layout_passes=False),
    scratch_types=[
        pltpu.VMEM((_CLASSES,), jnp.float32),
        pltpu.VMEM((_BATCH,), jnp.int32),
        pltpu.VMEM((_QB,), jnp.float32),
        pltpu.VMEM((_QB,), jnp.float32),
        pltpu.VMEM((_L,), jnp.float32),
        pltpu.SemaphoreType.DMA,
        pltpu.SemaphoreType.DMA,
        pltpu.SemaphoreType.DMA,
        pltpu.SemaphoreType.DMA,
    ],
)
def _center_partials(feat_hbm, lab_hbm, cent_hbm, out_hbm,
                     crow, labv, fq0, fq1, acc_v,
                     csem, fsem0, fsem1, lsem):
    wid = lax.axis_index("s") * _NC + lax.axis_index("c")
    j0 = wid * _RPW
    fqs = (fq0, fq1)
    fsems = (fsem0, fsem1)

    def issue_feat(r, q):
        return pltpu.async_copy(feat_hbm.at[j0 + r, pl.ds(q * _QB, _QB)],
                                fqs[q % 2], fsems[q % 2])

    lab_h = pltpu.async_copy(lab_hbm, labv, lsem)
    crow_h = pltpu.async_copy(cent_hbm.at[j0], crow, csem)
    pending = {(0, 0): issue_feat(0, 0), (0, 1): issue_feat(0, 1)}
    lab_h.wait()

    accs = tuple(jnp.zeros((_L,), jnp.float32) for _ in range(_UNROLL))
    for r in range(_RPW):
        crow_h.wait()
        for q in range(_NQ):
            fq = fqs[q % 2]
            pending.pop((r, q)).wait()

            def step(g, a, q=q, fq=fq):
                out = []
                for u in range(_UNROLL):
                    o = g * _L * _UNROLL + u * _L
                    idx = labv[pl.ds(q * _QB + o, _L)]
                    f = fq[pl.ds(o, _L)]
                    c = plsc.load_gather(crow, [idx])
                    d = f - c
                    out.append(a[u] + d * d)
                return tuple(out)

            accs = lax.fori_loop(0, _QB // (_L * _UNROLL), step, accs)

            if q + 2 < _NQ:
                pending[(r, q + 2)] = issue_feat(r, q + 2)
            elif r + 1 < _RPW:
                pending[(r + 1, q - 2)] = issue_feat(r + 1, q - 2)

        if r + 1 < _RPW:
            crow_h = pltpu.async_copy(cent_hbm.at[j0 + r + 1], crow, csem)

    acc_v[...] = functools.reduce(lambda x, y: x + y, accs)
    pltpu.sync_copy(acc_v, out_hbm.at[wid])


def kernel(features, labels, centers):
    lab = labels.astype(jnp.int32)
    partials = _center_partials(features.T, lab, centers.T)
    return jnp.sum(partials) / features.shape[0]
